# baseline (speedup 1.0000x reference)
import jax
import jax.numpy as jnp
from jax.experimental import pallas as pl
from jax.experimental.pallas import tpu as pltpu

_NUM_SCHEMAS = 16
_K_ACTIVE = 3
_BJ = 128
_BK = 2048


def _select_body(unit_ref, sw_ref, bias_ref, idx_ref, w_ref, selbias_ref):
    u = unit_ref[0]
    row = sw_ref[pl.ds(u, 1), :]
    iota = jax.lax.broadcasted_iota(jnp.int32, (1, _NUM_SCHEMAS), 1)
    vals = row
    keep = jnp.zeros_like(row)
    for a in range(_K_ACTIVE):
        m = jnp.max(vals)
        am = jnp.min(jnp.where(vals == m, iota, _NUM_SCHEMAS)).astype(jnp.int32)
        idx_ref[a] = am
        w_ref[a] = m
        hit = iota == am
        keep = jnp.where(hit, 1.0, keep)
        vals = jnp.where(hit, -jnp.inf, vals)
    wfull = row * keep
    selbias_ref[...] = jnp.dot(wfull, bias_ref[...],
                               preferred_element_type=jnp.float32)


def _combine_body(idx_ref, aw0_ref, aw1_ref, aw2_ref, w_ref, out_ref):
    out_ref[...] = (w_ref[0] * aw0_ref[0].T
                    + w_ref[1] * aw1_ref[0].T
                    + w_ref[2] * aw2_ref[0].T)


def kernel(all_weight, all_bias, schema_weighting, unit_idx):
    n_schemas, c_in, c_out = all_weight.shape
    unit = jnp.asarray(unit_idx, jnp.int32).reshape((1,))

    idx, w, selbias = pl.pallas_call(
        _select_body,
        in_specs=[
            pl.BlockSpec(memory_space=pltpu.SMEM),
            pl.BlockSpec(memory_space=pltpu.VMEM),
            pl.BlockSpec(memory_space=pltpu.VMEM),
        ],
        out_specs=[
            pl.BlockSpec(memory_space=pltpu.SMEM),
            pl.BlockSpec(memory_space=pltpu.SMEM),
            pl.BlockSpec(memory_space=pltpu.VMEM),
        ],
        out_shape=[
            jax.ShapeDtypeStruct((_K_ACTIVE,), jnp.int32),
            jax.ShapeDtypeStruct((_K_ACTIVE,), jnp.float32),
            jax.ShapeDtypeStruct((1, c_out), jnp.float32),
        ],
    )(unit, schema_weighting, all_bias)

    nk = c_out // _BK
    nj = c_in // _BJ
    grid_spec = pltpu.PrefetchScalarGridSpec(
        num_scalar_prefetch=1,
        grid=(nk, nj),
        in_specs=[
            pl.BlockSpec((1, _BJ, _BK), lambda k, j, idx_ref: (idx_ref[0], j, k)),
            pl.BlockSpec((1, _BJ, _BK), lambda k, j, idx_ref: (idx_ref[1], j, k)),
            pl.BlockSpec((1, _BJ, _BK), lambda k, j, idx_ref: (idx_ref[2], j, k)),
            pl.BlockSpec(memory_space=pltpu.SMEM),
        ],
        out_specs=pl.BlockSpec((_BK, _BJ), lambda k, j, idx_ref: (k, j)),
    )
    sel_weight = pl.pallas_call(
        _combine_body,
        grid_spec=grid_spec,
        out_shape=jax.ShapeDtypeStruct((c_out, c_in), jnp.float32),
    )(idx, all_weight, all_weight, all_weight, w)

    return sel_weight, selbias.reshape((c_out,))


# speedup vs baseline: 1.1359x; 1.1359x over previous
import jax
import jax.numpy as jnp
from jax.experimental import pallas as pl
from jax.experimental.pallas import tpu as pltpu

_NUM_SCHEMAS = 16
_K_ACTIVE = 3
_BJ = 512
_BK = 2048


def _select_body(unit_ref, sw_ref, bias_ref, idx_ref, w_ref, selbias_ref):
    u = unit_ref[0]
    row = sw_ref[pl.ds(u, 1), :]
    iota = jax.lax.broadcasted_iota(jnp.int32, (1, _NUM_SCHEMAS), 1)
    vals = row
    keep = jnp.zeros_like(row)
    for a in range(_K_ACTIVE):
        m = jnp.max(vals)
        am = jnp.min(jnp.where(vals == m, iota, _NUM_SCHEMAS)).astype(jnp.int32)
        idx_ref[a] = am
        w_ref[a] = m
        hit = iota == am
        keep = jnp.where(hit, 1.0, keep)
        vals = jnp.where(hit, -jnp.inf, vals)
    wfull = row * keep
    selbias_ref[...] = jnp.dot(wfull, bias_ref[...],
                               preferred_element_type=jnp.float32)


def _combine_body(idx_ref, aw0_ref, aw1_ref, aw2_ref, w_ref, out_ref):
    out_ref[...] = (w_ref[0] * aw0_ref[0].T
                    + w_ref[1] * aw1_ref[0].T
                    + w_ref[2] * aw2_ref[0].T)


def kernel(all_weight, all_bias, schema_weighting, unit_idx):
    n_schemas, c_in, c_out = all_weight.shape
    unit = jnp.asarray(unit_idx, jnp.int32).reshape((1,))

    idx, w, selbias = pl.pallas_call(
        _select_body,
        in_specs=[
            pl.BlockSpec(memory_space=pltpu.SMEM),
            pl.BlockSpec(memory_space=pltpu.VMEM),
            pl.BlockSpec(memory_space=pltpu.VMEM),
        ],
        out_specs=[
            pl.BlockSpec(memory_space=pltpu.SMEM),
            pl.BlockSpec(memory_space=pltpu.SMEM),
            pl.BlockSpec(memory_space=pltpu.VMEM),
        ],
        out_shape=[
            jax.ShapeDtypeStruct((_K_ACTIVE,), jnp.int32),
            jax.ShapeDtypeStruct((_K_ACTIVE,), jnp.float32),
            jax.ShapeDtypeStruct((1, c_out), jnp.float32),
        ],
    )(unit, schema_weighting, all_bias)

    nk = c_out // _BK
    nj = c_in // _BJ
    grid_spec = pltpu.PrefetchScalarGridSpec(
        num_scalar_prefetch=1,
        grid=(nk, nj),
        in_specs=[
            pl.BlockSpec((1, _BJ, _BK), lambda k, j, idx_ref: (idx_ref[0], j, k)),
            pl.BlockSpec((1, _BJ, _BK), lambda k, j, idx_ref: (idx_ref[1], j, k)),
            pl.BlockSpec((1, _BJ, _BK), lambda k, j, idx_ref: (idx_ref[2], j, k)),
            pl.BlockSpec(memory_space=pltpu.SMEM),
        ],
        out_specs=pl.BlockSpec((_BK, _BJ), lambda k, j, idx_ref: (k, j)),
    )
    sel_weight = pl.pallas_call(
        _combine_body,
        grid_spec=grid_spec,
        out_shape=jax.ShapeDtypeStruct((c_out, c_in), jnp.float32),
    )(idx, all_weight, all_weight, all_weight, w)

    return sel_weight, selbias.reshape((c_out,))
